# baseline (speedup 1.0000x reference)
import functools

import jax
import jax.numpy as jnp
from jax import lax
from jax.experimental import pallas as pl
from jax.experimental.pallas import tpu as pltpu
from jax.experimental.pallas import tpu_sc as plsc

ALPHA = 0.8
_NC = 2
_NS = 16
_NW = _NC * _NS

_CLS_CHUNKS = ((0, 6784), (6784, 6784), (13568, 6400))
_BOX_CHUNKS = ((0, 10112), (10112, 9856))
_CLS_W = 6784
_BOX_W = 10112
_TAIL = 19968


def _safe_pow(x, p):
    safe = jnp.where(x > 0, x, 1.0)
    return jnp.where(x > 0, jnp.exp(p * jnp.log(safe)), 0.0)


def _compute_body(cls3_ref, box2_ref, ctail_ref, btail_ref, idx_ref,
                  gt_ref, lab_ref, out_ref):
    n_tail = ctail_ref.shape[1]
    n_pos = idx_ref.shape[1]
    tail_slots = lax.broadcasted_iota(jnp.int32, (n_tail, n_pos), 0) + _TAIL
    tail_oh = (tail_slots == idx_ref[...]).astype(jnp.float32)
    cls_raw = (cls3_ref[0] + cls3_ref[1] + cls3_ref[2]
               + jnp.dot(ctail_ref[...], tail_oh,
                         preferred_element_type=jnp.float32,
                         precision=lax.Precision.HIGHEST))
    box_raw = (box2_ref[0][0:4, :] + box2_ref[1][0:4, :]
               + jnp.dot(btail_ref[...], tail_oh,
                         preferred_element_type=jnp.float32,
                         precision=lax.Precision.HIGHEST))
    sig = jax.nn.sigmoid(cls_raw)
    n_cls = cls_raw.shape[0]
    n_gt = lab_ref.shape[0]
    classes = lax.broadcasted_iota(jnp.int32, (n_gt, n_cls), 1)
    onehot = (classes == lab_ref[...]).astype(jnp.float32)
    cls_cost = jnp.dot(onehot, sig, preferred_element_type=jnp.float32,
                       precision=lax.Precision.HIGHEST)

    px1 = box_raw[0:1, :]
    py1 = box_raw[1:2, :]
    px2 = box_raw[2:3, :]
    py2 = box_raw[3:4, :]
    gx1 = gt_ref[:, 0:1]
    gy1 = gt_ref[:, 1:2]
    gx2 = gt_ref[:, 2:3]
    gy2 = gt_ref[:, 3:4]
    area_p = (px2 - px1) * (py2 - py1)
    area_g = (gx2 - gx1) * (gy2 - gy1)
    iw = jnp.clip(jnp.minimum(px2, gx2) - jnp.maximum(px1, gx1), 0.0)
    ih = jnp.clip(jnp.minimum(py2, gy2) - jnp.maximum(py1, gy1), 0.0)
    inter = iw * ih
    union = area_p + area_g - inter
    iou = inter / jnp.maximum(union, 1e-6)

    ov = _safe_pow(cls_cost, 1.0 - ALPHA) * _safe_pow(iou, ALPHA)
    q = jnp.max(ov, axis=0, keepdims=True)
    out_ref[0:1, :] = jnp.where(q < 0.0, 0.0, q)
    out_ref[1:2, :] = jnp.ones_like(q)


def _tc_scatter_body(q_ref, idxc_ref, out_ref):
    nb = out_ref.shape[1]
    n_pos = idxc_ref.shape[0]
    n0 = pl.program_id(0) * nb
    cols = lax.broadcasted_iota(jnp.int32, (n_pos, nb), 1) + n0
    oh = (cols == idxc_ref[...]).astype(jnp.float32)
    sc = jnp.dot(q_ref[...], oh, preferred_element_type=jnp.float32,
                 precision=lax.Precision.HIGHEST)
    s = sc[0:1, :]
    cnt = sc[1:2, :]
    out_ref[...] = jnp.where(cnt > 0, s / jnp.maximum(cnt, 1.0), 0.0)


def _make_gather(n_pos, n_cls):
    mesh = plsc.VectorSubcoreMesh(core_axis_name="c", subcore_axis_name="s")
    n_grp = n_cls // 8
    n_cchunk = len(_CLS_CHUNKS)
    n_cls_workers = n_grp * n_cchunk

    @functools.partial(
        pl.kernel,
        out_type=[
            jax.ShapeDtypeStruct((n_cchunk, n_cls, n_pos), jnp.float32),
            jax.ShapeDtypeStruct((len(_BOX_CHUNKS), 8, n_pos), jnp.float32),
        ],
        mesh=mesh,
        scratch_types=[
            pltpu.VMEM((8, _CLS_W), jnp.float32),
            pltpu.VMEM((4, _BOX_W), jnp.float32),
            pltpu.VMEM((n_pos,), jnp.int32),
            pltpu.VMEM((8, n_pos), jnp.float32),
        ],
        compiler_params=pltpu.CompilerParams(needs_layout_passes=False),
    )
    def gather_k(cls_hbm, box_hbm, idx_hbm, cls_out, box_out,
                 cstage_v, bstage_v, idx_v, vals_v):
        wid = lax.axis_index("s") * _NC + lax.axis_index("c")
        pltpu.sync_copy(idx_hbm, idx_v)
        is_cls = wid < n_cls_workers
        rvecs = [jnp.full((16,), r, jnp.int32) for r in range(8)]
        zero = jnp.zeros((16,), jnp.float32)

        def gather_loop(stage, off, width, n_rows):
            def body(j, _):
                iv = idx_v[pl.ds(j * 16, 16)]
                loc = iv - off
                m = (loc >= 0) & (loc < width)
                locc = jnp.where(m, loc, 0)
                for r in range(n_rows):
                    v = plsc.load_gather(stage, [rvecs[r], locc])
                    vals_v[r, pl.ds(j * 16, 16)] = jnp.where(m, v, zero)
                return 0

            lax.fori_loop(0, n_pos // 16, body, 0, unroll=2)

        @pl.when(is_cls)
        def _():
            g = wid // n_cchunk
            k = wid % n_cchunk
            for kk, (off, width) in enumerate(_CLS_CHUNKS):
                @pl.when(k == kk)
                def _():
                    pltpu.sync_copy(
                        cls_hbm.at[pl.ds(g * 8, 8), pl.ds(off, width)],
                        cstage_v.at[:, pl.ds(0, width)])
            off_t = k * _CLS_CHUNKS[0][1]
            width_t = jnp.where(k == n_cchunk - 1,
                                _CLS_CHUNKS[-1][1], _CLS_CHUNKS[0][1])
            gather_loop(cstage_v, off_t, width_t, 8)
            pltpu.sync_copy(vals_v, cls_out.at[k, pl.ds(g * 8, 8)])

        @pl.when(jnp.logical_not(is_cls))
        def _():
            h = wid - n_cls_workers
            for hh, (off, width) in enumerate(_BOX_CHUNKS):
                @pl.when(h == hh)
                def _():
                    pltpu.sync_copy(
                        box_hbm.at[:, pl.ds(off, width)],
                        bstage_v.at[:, pl.ds(0, width)])
            off_t = h * _BOX_CHUNKS[0][1]
            width_t = jnp.where(h == len(_BOX_CHUNKS) - 1,
                                _BOX_CHUNKS[-1][1], _BOX_CHUNKS[0][1])
            gather_loop(bstage_v, off_t, width_t, 4)
            pltpu.sync_copy(vals_v, box_out.at[h])

    return gather_k


def _make_scatter(n_pos, n):
    granules = n // 8
    c_lo = (granules // _NW) * 8
    c_hi = c_lo + 8
    n_hi = granules % _NW
    buf = ((c_hi + 15) // 16) * 16
    mesh = plsc.VectorSubcoreMesh(core_axis_name="c", subcore_axis_name="s")

    @functools.partial(
        pl.kernel,
        out_type=jax.ShapeDtypeStruct((n,), jnp.float32),
        mesh=mesh,
        scratch_types=[
            pltpu.VMEM((buf,), jnp.float32),
            pltpu.VMEM((n_pos,), jnp.int32),
            pltpu.VMEM((n_pos,), jnp.float32),
        ],
        compiler_params=pltpu.CompilerParams(
            use_tc_tiling_on_sc=False, needs_layout_passes=False),
    )
    def scatter_k(idx_hbm, q_hbm, out_hbm, chunk_v, idx_v, q_v):
        wid = lax.axis_index("s") * _NC + lax.axis_index("c")
        in_hi = wid < n_hi
        off = jnp.where(in_hi, wid * c_hi,
                        n_hi * c_hi + (wid - n_hi) * c_lo)
        size = jnp.where(in_hi, c_hi, c_lo)
        zeros16 = jnp.zeros((16,), jnp.float32)
        for j in range(buf // 16):
            chunk_v[pl.ds(j * 16, 16)] = zeros16
        pltpu.sync_copy(idx_hbm, idx_v)
        pltpu.sync_copy(q_hbm, q_v)
        for j in range(n_pos // 16):
            iv = idx_v[pl.ds(j * 16, 16)]
            qv = q_v[pl.ds(j * 16, 16)]
            m = (iv >= off) & (iv < off + size)
            loc = jnp.where(m, iv - off, 0)
            plsc.store_scatter(chunk_v, [loc], qv, mask=m)

        @pl.when(in_hi)
        def _():
            pltpu.sync_copy(chunk_v.at[pl.ds(0, c_hi)],
                            out_hbm.at[pl.ds(off, c_hi)])

        @pl.when(jnp.logical_not(in_hi))
        def _():
            pltpu.sync_copy(chunk_v.at[pl.ds(0, c_lo)],
                            out_hbm.at[pl.ds(off, c_lo)])

    return scatter_k


def kernel(bboxes, cls_scores, bbox_preds, gt_bboxes, bbox_levels,
           positive_inds, gt_labels):
    n = bboxes.shape[0]
    n_cls = cls_scores.shape[1]
    n_pos = positive_inds.shape[0]

    idx = positive_inds.astype(jnp.int32)
    cls_t = cls_scores.astype(jnp.float32).T
    box_t = bbox_preds.astype(jnp.float32).T

    cls3, box2 = _make_gather(n_pos, n_cls)(cls_t, box_t, idx)

    cls_tail = cls_t[:, _TAIL:]
    box_tail = box_t[:, _TAIL:]
    lab = gt_labels.astype(jnp.int32).reshape(-1, 1)
    q2 = pl.pallas_call(
        _compute_body,
        out_shape=jax.ShapeDtypeStruct((2, n_pos), jnp.float32),
    )(cls3, box2, cls_tail, box_tail, idx.reshape(1, -1),
      gt_bboxes.astype(jnp.float32), lab)

    nb = 2048
    grid = (n + nb - 1) // nb
    out = pl.pallas_call(
        _tc_scatter_body,
        grid=(grid,),
        in_specs=[
            pl.BlockSpec((2, n_pos), lambda i: (0, 0)),
            pl.BlockSpec((n_pos, 1), lambda i: (0, 0)),
        ],
        out_specs=pl.BlockSpec((1, nb), lambda i: (0, i)),
        out_shape=jax.ShapeDtypeStruct((1, n), jnp.float32),
    )(q2, idx.reshape(-1, 1))
    return out.reshape(n)


# speedup vs baseline: 1.6421x; 1.6421x over previous
import functools

import jax
import jax.numpy as jnp
from jax import lax
from jax.experimental import pallas as pl
from jax.experimental.pallas import tpu as pltpu
from jax.experimental.pallas import tpu_sc as plsc

ALPHA = 0.8
_NC = 2
_NS = 16
_NW = _NC * _NS

_CLS_CHUNKS = ((0, 6784), (6784, 6784), (13568, 6400))
_BOX_CHUNKS = ((0, 10112), (10112, 9856))
_CLS_W = 6784
_BOX_W = 10112
_TAIL = 19968


def _safe_pow(x, p):
    safe = jnp.where(x > 0, x, 1.0)
    return jnp.where(x > 0, jnp.exp(p * jnp.log(safe)), 0.0)


def _compute_body(cls3_ref, box2_ref, ctail_ref, btail_ref, idx_ref,
                  gt_ref, lab_ref, out_ref):
    n_tail = ctail_ref.shape[1]
    n_pos = idx_ref.shape[1]
    tail_slots = lax.broadcasted_iota(jnp.int32, (n_tail, n_pos), 0) + _TAIL
    tail_oh = (tail_slots == idx_ref[...]).astype(jnp.float32)
    cls_raw = (cls3_ref[0] + cls3_ref[1] + cls3_ref[2]
               + jnp.dot(ctail_ref[...], tail_oh,
                         preferred_element_type=jnp.float32,
                         precision=lax.Precision.HIGHEST))
    box_raw = (box2_ref[0][0:4, :] + box2_ref[1][0:4, :]
               + jnp.dot(btail_ref[...], tail_oh,
                         preferred_element_type=jnp.float32,
                         precision=lax.Precision.HIGHEST))
    sig = jax.nn.sigmoid(cls_raw)
    n_cls = cls_raw.shape[0]
    n_gt = lab_ref.shape[0]
    classes = lax.broadcasted_iota(jnp.int32, (n_gt, n_cls), 1)
    onehot = (classes == lab_ref[...]).astype(jnp.float32)
    cls_cost = jnp.dot(onehot, sig, preferred_element_type=jnp.float32,
                       precision=lax.Precision.HIGHEST)

    px1 = box_raw[0:1, :]
    py1 = box_raw[1:2, :]
    px2 = box_raw[2:3, :]
    py2 = box_raw[3:4, :]
    gx1 = gt_ref[:, 0:1]
    gy1 = gt_ref[:, 1:2]
    gx2 = gt_ref[:, 2:3]
    gy2 = gt_ref[:, 3:4]
    area_p = (px2 - px1) * (py2 - py1)
    area_g = (gx2 - gx1) * (gy2 - gy1)
    iw = jnp.clip(jnp.minimum(px2, gx2) - jnp.maximum(px1, gx1), 0.0)
    ih = jnp.clip(jnp.minimum(py2, gy2) - jnp.maximum(py1, gy1), 0.0)
    inter = iw * ih
    union = area_p + area_g - inter
    iou = inter / jnp.maximum(union, 1e-6)

    ov = _safe_pow(cls_cost, 1.0 - ALPHA) * _safe_pow(iou, ALPHA)
    q = jnp.max(ov, axis=0, keepdims=True)
    out_ref[...] = jnp.where(q < 0.0, 0.0, q)


def _make_gather(n_pos, n_cls):
    mesh = plsc.VectorSubcoreMesh(core_axis_name="c", subcore_axis_name="s")
    n_grp = n_cls // 8
    n_cchunk = len(_CLS_CHUNKS)
    n_cls_workers = n_grp * n_cchunk

    @functools.partial(
        pl.kernel,
        out_type=[
            jax.ShapeDtypeStruct((n_cchunk, n_cls, n_pos), jnp.float32),
            jax.ShapeDtypeStruct((len(_BOX_CHUNKS), 8, n_pos), jnp.float32),
        ],
        mesh=mesh,
        scratch_types=[
            pltpu.VMEM((8, _CLS_W), jnp.float32),
            pltpu.VMEM((4, _BOX_W), jnp.float32),
            pltpu.VMEM((n_pos,), jnp.int32),
            pltpu.VMEM((8, n_pos), jnp.float32),
        ],
        compiler_params=pltpu.CompilerParams(needs_layout_passes=False),
    )
    def gather_k(cls_hbm, box_hbm, idx_hbm, cls_out, box_out,
                 cstage_v, bstage_v, idx_v, vals_v):
        wid = lax.axis_index("s") * _NC + lax.axis_index("c")
        pltpu.sync_copy(idx_hbm, idx_v)
        is_cls = wid < n_cls_workers
        rvecs = [jnp.full((16,), r, jnp.int32) for r in range(8)]
        zero = jnp.zeros((16,), jnp.float32)

        def gather_loop(stage, off, width, n_rows):
            def body(j, _):
                iv = idx_v[pl.ds(j * 16, 16)]
                loc = iv - off
                m = (loc >= 0) & (loc < width)
                locc = jnp.where(m, loc, 0)
                for r in range(n_rows):
                    v = plsc.load_gather(stage, [rvecs[r], locc])
                    vals_v[r, pl.ds(j * 16, 16)] = jnp.where(m, v, zero)
                return 0

            lax.fori_loop(0, n_pos // 16, body, 0, unroll=2)

        @pl.when(is_cls)
        def _():
            g = wid // n_cchunk
            k = wid % n_cchunk
            for kk, (off, width) in enumerate(_CLS_CHUNKS):
                @pl.when(k == kk)
                def _():
                    pltpu.sync_copy(
                        cls_hbm.at[pl.ds(g * 8, 8), pl.ds(off, width)],
                        cstage_v.at[:, pl.ds(0, width)])
            off_t = k * _CLS_CHUNKS[0][1]
            width_t = jnp.where(k == n_cchunk - 1,
                                _CLS_CHUNKS[-1][1], _CLS_CHUNKS[0][1])
            gather_loop(cstage_v, off_t, width_t, 8)
            pltpu.sync_copy(vals_v, cls_out.at[k, pl.ds(g * 8, 8)])

        @pl.when(jnp.logical_not(is_cls))
        def _():
            h = wid - n_cls_workers
            for hh, (off, width) in enumerate(_BOX_CHUNKS):
                @pl.when(h == hh)
                def _():
                    pltpu.sync_copy(
                        box_hbm.at[:, pl.ds(off, width)],
                        bstage_v.at[:, pl.ds(0, width)])
            off_t = h * _BOX_CHUNKS[0][1]
            width_t = jnp.where(h == len(_BOX_CHUNKS) - 1,
                                _BOX_CHUNKS[-1][1], _BOX_CHUNKS[0][1])
            gather_loop(bstage_v, off_t, width_t, 4)
            pltpu.sync_copy(vals_v, box_out.at[h])

    return gather_k


def _make_scatter(n_pos, n):
    granules = n // 8
    c_lo = (granules // _NW) * 8
    c_hi = c_lo + 8
    n_hi = granules % _NW
    buf = ((c_hi + 15) // 16) * 16
    mesh = plsc.VectorSubcoreMesh(core_axis_name="c", subcore_axis_name="s")

    @functools.partial(
        pl.kernel,
        out_type=jax.ShapeDtypeStruct((n,), jnp.float32),
        mesh=mesh,
        scratch_types=[
            pltpu.VMEM((buf,), jnp.float32),
            pltpu.VMEM((n_pos,), jnp.int32),
            pltpu.VMEM((n_pos,), jnp.float32),
        ],
        compiler_params=pltpu.CompilerParams(
            use_tc_tiling_on_sc=False, needs_layout_passes=False),
    )
    def scatter_k(idx_hbm, q_hbm, out_hbm, chunk_v, idx_v, q_v):
        wid = lax.axis_index("s") * _NC + lax.axis_index("c")
        in_hi = wid < n_hi
        off = jnp.where(in_hi, wid * c_hi,
                        n_hi * c_hi + (wid - n_hi) * c_lo)
        size = jnp.where(in_hi, c_hi, c_lo)
        zeros16 = jnp.zeros((16,), jnp.float32)
        for j in range(buf // 16):
            chunk_v[pl.ds(j * 16, 16)] = zeros16
        pltpu.sync_copy(idx_hbm, idx_v)
        pltpu.sync_copy(q_hbm, q_v)
        for j in range(n_pos // 16):
            iv = idx_v[pl.ds(j * 16, 16)]
            qv = q_v[pl.ds(j * 16, 16)]
            m = (iv >= off) & (iv < off + size)
            loc = jnp.where(m, iv - off, 0)
            plsc.store_scatter(chunk_v, [loc], qv, mask=m)

        @pl.when(in_hi)
        def _():
            pltpu.sync_copy(chunk_v.at[pl.ds(0, c_hi)],
                            out_hbm.at[pl.ds(off, c_hi)])

        @pl.when(jnp.logical_not(in_hi))
        def _():
            pltpu.sync_copy(chunk_v.at[pl.ds(0, c_lo)],
                            out_hbm.at[pl.ds(off, c_lo)])

    return scatter_k


def kernel(bboxes, cls_scores, bbox_preds, gt_bboxes, bbox_levels,
           positive_inds, gt_labels):
    n = bboxes.shape[0]
    n_cls = cls_scores.shape[1]
    n_pos = positive_inds.shape[0]

    idx = positive_inds.astype(jnp.int32)
    cls_t = cls_scores.astype(jnp.float32).T
    box_t = bbox_preds.astype(jnp.float32).T

    cls3, box2 = _make_gather(n_pos, n_cls)(cls_t, box_t, idx)

    cls_tail = cls_t[:, _TAIL:]
    box_tail = box_t[:, _TAIL:]
    lab = gt_labels.astype(jnp.int32).reshape(-1, 1)
    q = pl.pallas_call(
        _compute_body,
        out_shape=jax.ShapeDtypeStruct((1, n_pos), jnp.float32),
    )(cls3, box2, cls_tail, box_tail, idx.reshape(1, -1),
      gt_bboxes.astype(jnp.float32), lab)

    return _make_scatter(n_pos, n)(idx, q.reshape(-1))
